# baseline (speedup 1.0000x reference)
import functools
import math

import jax
import jax.numpy as jnp
from jax import lax
from jax.experimental import pallas as pl
from jax.experimental.pallas import tpu as pltpu
from jax.experimental.pallas import tpu_sc as plsc

_VOCAB = 1000000
_EMBED = 64
_NUM_SAMPLED = 8192
_BATCH = 4096

_NC = 2
_NS = 16
_NW = _NC * _NS
_CHUNK = 128
_L = 16

_SAMP_PER_W = _NUM_SAMPLED // _NW
_TRUE_PER_W = _BATCH // _NW
_SAMP_CHUNKS = _SAMP_PER_W // _CHUNK
_TRUE_CHUNKS = _TRUE_PER_W // _CHUNK

_BROWS = -(-_VOCAB // 128)

_TB = 512
_TGRID = -(-_VOCAB // _TB)
_TROWS = _TGRID * (_TB // 2)

_INV_LOG_V1 = 1.0 / math.log(float(_VOCAB) + 1.0)


def _transpose_body(x_ref, o_ref):
    x = x_ref[...]
    eye = (lax.broadcasted_iota(jnp.int32, (_EMBED, _EMBED), 0)
           == lax.broadcasted_iota(jnp.int32, (_EMBED, _EMBED), 1)
           ).astype(jnp.float32)
    half = _TB // 2
    tl = lax.dot_general(x[:, :half], eye, (((0,), (0,)), ((), ())),
                         preferred_element_type=jnp.float32)
    tr = lax.dot_general(x[:, half:], eye, (((0,), (0,)), ((), ())),
                         preferred_element_type=jnp.float32)
    o_ref[...] = jnp.concatenate([tl, tr], axis=1)


_transpose = pl.pallas_call(
    _transpose_body,
    grid=(_TGRID,),
    in_specs=[pl.BlockSpec((_EMBED, _TB), lambda j: (0, j))],
    out_specs=pl.BlockSpec((_TB // 2, 128), lambda j: (j, 0)),
    out_shape=jax.ShapeDtypeStruct((_TROWS, 128), jnp.float32),
)


def _sc_gather_body(w_hbm, b_hbm, sidx_hbm, tidx_hbm,
                    sw_hbm, sb_hbm, tw_hbm, tb_hbm,
                    idx_s, idx_t, idxw_s, idxw_t, idxb_s, idxb_t,
                    rows_s, rows_t, brows_s, brows_t, bs_v, bt_v, sem):
    wid = lax.axis_index("s") * _NC + lax.axis_index("c")
    pltpu.sync_copy(sidx_hbm.at[pl.ds(wid * _SAMP_CHUNKS, _SAMP_CHUNKS)], idx_s)
    pltpu.sync_copy(tidx_hbm.at[pl.ds(wid * _TRUE_CHUNKS, _TRUE_CHUNKS)], idx_t)
    for j in range(_SAMP_CHUNKS):
        for g in range(_CHUNK // _L):
            v = idx_s[j, pl.ds(g * _L, _L)]
            idxw_s[j, pl.ds(g * _L, _L)] = (
                lax.shift_left(lax.shift_right_logical(v, 9), 8)
                + jnp.bitwise_and(v, _TB // 2 - 1))
            idxb_s[j, pl.ds(g * _L, _L)] = lax.shift_right_logical(v, 7)
    for j in range(_TRUE_CHUNKS):
        for g in range(_CHUNK // _L):
            v = idx_t[j, pl.ds(g * _L, _L)]
            idxw_t[j, pl.ds(g * _L, _L)] = (
                lax.shift_left(lax.shift_right_logical(v, 9), 8)
                + jnp.bitwise_and(v, _TB // 2 - 1))
            idxb_t[j, pl.ds(g * _L, _L)] = lax.shift_right_logical(v, 7)
    copies = []
    for j in range(_SAMP_CHUNKS):
        copies.append(pltpu.async_copy(
            w_hbm.at[idxw_s.at[j]], rows_s.at[pl.ds(j * _CHUNK, _CHUNK)], sem))
        copies.append(pltpu.async_copy(
            b_hbm.at[idxb_s.at[j]], brows_s.at[pl.ds(j * _CHUNK, _CHUNK)], sem))
    for j in range(_TRUE_CHUNKS):
        copies.append(pltpu.async_copy(
            w_hbm.at[idxw_t.at[j]], rows_t.at[pl.ds(j * _CHUNK, _CHUNK)], sem))
        copies.append(pltpu.async_copy(
            b_hbm.at[idxb_t.at[j]], brows_t.at[pl.ds(j * _CHUNK, _CHUNK)], sem))
    for c in copies:
        c.wait()
    for j in range(_SAMP_CHUNKS):
        for g in range(_CHUNK // _L):
            pos = j * _CHUNK + g * _L
            lane = jnp.bitwise_and(idx_s[j, pl.ds(g * _L, _L)], 127)
            row = lax.iota(jnp.int32, _L) + pos
            bs_v[pl.ds(pos, _L)] = plsc.load_gather(brows_s, [row, lane])
    for j in range(_TRUE_CHUNKS):
        for g in range(_CHUNK // _L):
            pos = j * _CHUNK + g * _L
            lane = jnp.bitwise_and(idx_t[j, pl.ds(g * _L, _L)], 127)
            row = lax.iota(jnp.int32, _L) + pos
            bt_v[pl.ds(pos, _L)] = plsc.load_gather(brows_t, [row, lane])
    pltpu.sync_copy(rows_s, sw_hbm.at[pl.ds(wid * _SAMP_PER_W, _SAMP_PER_W)])
    pltpu.sync_copy(bs_v, sb_hbm.at[pl.ds(wid * _SAMP_PER_W, _SAMP_PER_W)])
    pltpu.sync_copy(rows_t, tw_hbm.at[pl.ds(wid * _TRUE_PER_W, _TRUE_PER_W)])
    pltpu.sync_copy(bt_v, tb_hbm.at[pl.ds(wid * _TRUE_PER_W, _TRUE_PER_W)])


@functools.cache
def _sc_gather():
  return pl.kernel(
    _sc_gather_body,
    out_type=(
        jax.ShapeDtypeStruct((_NUM_SAMPLED, 128), jnp.float32),
        jax.ShapeDtypeStruct((_NUM_SAMPLED,), jnp.float32),
        jax.ShapeDtypeStruct((_BATCH, 128), jnp.float32),
        jax.ShapeDtypeStruct((_BATCH,), jnp.float32),
    ),
    mesh=plsc.VectorSubcoreMesh(core_axis_name="c", subcore_axis_name="s"),
    scratch_types=[
        pltpu.VMEM((_SAMP_CHUNKS, _CHUNK), jnp.int32),
        pltpu.VMEM((_TRUE_CHUNKS, _CHUNK), jnp.int32),
        pltpu.VMEM((_SAMP_CHUNKS, _CHUNK), jnp.int32),
        pltpu.VMEM((_TRUE_CHUNKS, _CHUNK), jnp.int32),
        pltpu.VMEM((_SAMP_CHUNKS, _CHUNK), jnp.int32),
        pltpu.VMEM((_TRUE_CHUNKS, _CHUNK), jnp.int32),
        pltpu.VMEM((_SAMP_PER_W, 128), jnp.float32),
        pltpu.VMEM((_TRUE_PER_W, 128), jnp.float32),
        pltpu.VMEM((_SAMP_PER_W, 128), jnp.float32),
        pltpu.VMEM((_TRUE_PER_W, 128), jnp.float32),
        pltpu.VMEM((_SAMP_PER_W,), jnp.float32),
        pltpu.VMEM((_TRUE_PER_W,), jnp.float32),
        pltpu.SemaphoreType.DMA,
    ],
    compiler_params=pltpu.CompilerParams(
        use_tc_tiling_on_sc=True, needs_layout_passes=False),
  )


_BT = 256
_GRID = _BATCH // _BT


def _log1p_small(x):
    return x * (1.0 - x * (1 / 2 - x * (1 / 3 - x * (1 / 4 - x * (
        1 / 5 - x * (1 / 6 - x * (1 / 7)))))))


def _expm1_neg(y):
    small = y * (1.0 + y * (1 / 2 + y * (1 / 6 + y * (1 / 24 + y * (1 / 120)))))
    return jnp.where(y > -0.03, small, jnp.exp(y) - 1.0)


def _log_expected(idsf):
    p = jnp.log((idsf + 2.0) / (idsf + 1.0)) * _INV_LOG_V1
    return jnp.log(-_expm1_neg(_NUM_SAMPLED * _log1p_small(-p)))


def _half_select(rows, ids_col):
    hi = jnp.bitwise_and(ids_col, _TB // 2) != 0
    return jnp.where(hi, rows[:, _EMBED:], rows[:, :_EMBED])


def _tc_loss_body(pred_ref, tw_ref, tb_ref, lab_ref, sw_ref, sb_ref, sid_ref,
                  sidc_ref, out_ref):
    i = pl.program_id(0)
    pred = pred_ref[...]
    sw = _half_select(sw_ref[...], sidc_ref[...])
    logits = lax.dot_general(
        pred, sw, (((1,), (1,)), ((), ())),
        preferred_element_type=jnp.float32)
    sidf = sid_ref[...].astype(jnp.float32)
    logits = logits + (sb_ref[...] - _log_expected(sidf))
    labf = lab_ref[...].astype(jnp.float32)
    tw = _half_select(tw_ref[...], lab_ref[...])
    tl = (jnp.sum(pred * tw, axis=1, keepdims=True)
          + tb_ref[...] - _log_expected(labf))
    m = jnp.maximum(jnp.max(logits, axis=1, keepdims=True), tl)
    se = (jnp.sum(jnp.exp(logits - m), axis=1, keepdims=True)
          + jnp.exp(tl - m))
    per_ex = m + jnp.log(se) - tl
    part = jnp.sum(per_ex) * (1.0 / _BATCH)

    @pl.when(i == 0)
    def _():
        out_ref[0, 0] = 0.0

    out_ref[0, 0] += part


_tc_loss = pl.pallas_call(
    _tc_loss_body,
    grid=(_GRID,),
    in_specs=[
        pl.BlockSpec((_BT, _EMBED), lambda i: (i, 0)),
        pl.BlockSpec((_BT, 128), lambda i: (i, 0)),
        pl.BlockSpec((_BT, 1), lambda i: (i, 0)),
        pl.BlockSpec((_BT, 1), lambda i: (i, 0)),
        pl.BlockSpec((_NUM_SAMPLED, 128), lambda i: (0, 0)),
        pl.BlockSpec((1, _NUM_SAMPLED), lambda i: (0, 0)),
        pl.BlockSpec((1, _NUM_SAMPLED), lambda i: (0, 0)),
        pl.BlockSpec((_NUM_SAMPLED, 1), lambda i: (0, 0)),
    ],
    out_specs=pl.BlockSpec(memory_space=pltpu.SMEM),
    out_shape=jax.ShapeDtypeStruct((1, 1), jnp.float32),
)


def kernel(predictions, labels, weights, biases):
    u = jax.random.uniform(jax.random.key(42), (_NUM_SAMPLED,), dtype=jnp.float32)
    sampled = jnp.clip(
        (jnp.exp(u * jnp.log(_VOCAB + 1.0)) - 1.0).astype(jnp.int32),
        0, _VOCAB - 1)

    labels_i32 = labels.astype(jnp.int32).reshape(-1)
    bias_rows = jnp.pad(biases, (0, _BROWS * 128 - _VOCAB)).reshape(_BROWS, 128)
    wtab = _transpose(weights.T)
    samp_w, samp_b, true_w, true_b = _sc_gather()(
        wtab,
        bias_rows,
        sampled.reshape(_NUM_SAMPLED // _CHUNK, _CHUNK),
        labels_i32.reshape(_BATCH // _CHUNK, _CHUNK),
    )
    loss = _tc_loss(
        predictions,
        true_w,
        true_b.reshape(_BATCH, 1),
        labels_i32.reshape(_BATCH, 1),
        samp_w,
        samp_b.reshape(1, _NUM_SAMPLED),
        sampled.reshape(1, _NUM_SAMPLED),
        sampled.reshape(_NUM_SAMPLED, 1),
    )
    return loss[0, 0]


# speedup vs baseline: 2.1672x; 2.1672x over previous
import functools
import math

import jax
import jax.numpy as jnp
from jax import lax
from jax.experimental import pallas as pl
from jax.experimental.pallas import tpu as pltpu
from jax.experimental.pallas import tpu_sc as plsc

_VOCAB = 1000000
_EMBED = 64
_NUM_SAMPLED = 8192
_BATCH = 4096

_NC = 2
_NS = 16
_NW = _NC * _NS
_CHUNK = 128
_L = 16

_SAMP_PER_W = _NUM_SAMPLED // _NW
_TRUE_PER_W = _BATCH // _NW
_SAMP_CHUNKS = _SAMP_PER_W // _CHUNK
_TRUE_CHUNKS = _TRUE_PER_W // _CHUNK

_TB = 2048
_TGRID = -(-_VOCAB // _TB)
_TROWS = _TGRID * _TB

_INV_LOG_V1 = 1.0 / math.log(float(_VOCAB) + 1.0)


def _transpose_body(x_ref, b_ref, o_ref):
    x = x_ref[...]
    eye = (lax.broadcasted_iota(jnp.int32, (_EMBED, _EMBED), 0)
           == lax.broadcasted_iota(jnp.int32, (_EMBED, _EMBED), 1)
           ).astype(jnp.float32)
    xt = lax.dot_general(x, eye, (((0,), (0,)), ((), ())),
                         preferred_element_type=jnp.float32)
    bcol = lax.dot_general(b_ref[...], jnp.ones((1, 1), jnp.float32),
                           (((0,), (0,)), ((), ())),
                           preferred_element_type=jnp.float32)
    o_ref[...] = jnp.concatenate(
        [xt, bcol, jnp.zeros((_TB, 63), jnp.float32)], axis=1)


_transpose = pl.pallas_call(
    _transpose_body,
    grid=(_TGRID,),
    in_specs=[
        pl.BlockSpec((_EMBED, _TB), lambda j: (0, j)),
        pl.BlockSpec((1, _TB), lambda j: (0, j)),
    ],
    out_specs=pl.BlockSpec((_TB, 128), lambda j: (j, 0)),
    out_shape=jax.ShapeDtypeStruct((_TROWS, 128), jnp.float32),
)


def _sc_gather_body(w_hbm, sidx_hbm, tidx_hbm,
                    sw_hbm, tw_hbm,
                    idx_s, idx_t, rows_s, rows_t, sem):
    wid = lax.axis_index("s") * _NC + lax.axis_index("c")
    pltpu.sync_copy(sidx_hbm.at[pl.ds(wid * _SAMP_CHUNKS, _SAMP_CHUNKS)], idx_s)
    pltpu.sync_copy(tidx_hbm.at[pl.ds(wid * _TRUE_CHUNKS, _TRUE_CHUNKS)], idx_t)
    copies = []
    for j in range(_SAMP_CHUNKS):
        copies.append(pltpu.async_copy(
            w_hbm.at[idx_s.at[j]], rows_s.at[pl.ds(j * _CHUNK, _CHUNK)], sem))
    for j in range(_TRUE_CHUNKS):
        copies.append(pltpu.async_copy(
            w_hbm.at[idx_t.at[j]], rows_t.at[pl.ds(j * _CHUNK, _CHUNK)], sem))
    for c in copies:
        c.wait()
    pltpu.sync_copy(rows_s, sw_hbm.at[pl.ds(wid * _SAMP_PER_W, _SAMP_PER_W)])
    pltpu.sync_copy(rows_t, tw_hbm.at[pl.ds(wid * _TRUE_PER_W, _TRUE_PER_W)])


@functools.cache
def _sc_gather():
  return pl.kernel(
    _sc_gather_body,
    out_type=(
        jax.ShapeDtypeStruct((_NUM_SAMPLED, 128), jnp.float32),
        jax.ShapeDtypeStruct((_BATCH, 128), jnp.float32),
    ),
    mesh=plsc.VectorSubcoreMesh(core_axis_name="c", subcore_axis_name="s"),
    scratch_types=[
        pltpu.VMEM((_SAMP_CHUNKS, _CHUNK), jnp.int32),
        pltpu.VMEM((_TRUE_CHUNKS, _CHUNK), jnp.int32),
        pltpu.VMEM((_SAMP_PER_W, 128), jnp.float32),
        pltpu.VMEM((_TRUE_PER_W, 128), jnp.float32),
        pltpu.SemaphoreType.DMA,
    ],
    compiler_params=pltpu.CompilerParams(
        use_tc_tiling_on_sc=True, needs_layout_passes=False),
  )


_BT = 256
_GRID = _BATCH // _BT


def _log1p_small(x):
    return x * (1.0 - x * (1 / 2 - x * (1 / 3 - x * (1 / 4 - x * (
        1 / 5 - x * (1 / 6 - x * (1 / 7)))))))


def _expm1_neg(y):
    small = y * (1.0 + y * (1 / 2 + y * (1 / 6 + y * (1 / 24 + y * (1 / 120)))))
    return jnp.where(y > -0.03, small, jnp.exp(y) - 1.0)


def _log_expected(idsf):
    p = jnp.log((idsf + 2.0) / (idsf + 1.0)) * _INV_LOG_V1
    return jnp.log(-_expm1_neg(_NUM_SAMPLED * _log1p_small(-p)))


def _tc_loss_body(pred_ref, tw_ref, lab_ref, sw_ref, sid_ref, out_ref):
    i = pl.program_id(0)
    pred = pred_ref[...]
    pred_aug = jnp.concatenate(
        [pred, jnp.ones((_BT, 1), jnp.float32)], axis=1)
    sw = sw_ref[...][:, :_EMBED + 1]
    logits = lax.dot_general(
        pred_aug, sw, (((1,), (1,)), ((), ())),
        preferred_element_type=jnp.float32)
    sidf = sid_ref[...].astype(jnp.float32)
    logits = logits - _log_expected(sidf)
    labf = lab_ref[...].astype(jnp.float32)
    tw = tw_ref[...][:, :_EMBED + 1]
    tl = (jnp.sum(pred_aug * tw, axis=1, keepdims=True)
          - _log_expected(labf))
    m = jnp.maximum(jnp.max(logits, axis=1, keepdims=True), tl)
    se = (jnp.sum(jnp.exp(logits - m), axis=1, keepdims=True)
          + jnp.exp(tl - m))
    per_ex = m + jnp.log(se) - tl
    part = jnp.sum(per_ex) * (1.0 / _BATCH)

    @pl.when(i == 0)
    def _():
        out_ref[0, 0] = 0.0

    out_ref[0, 0] += part


_tc_loss = pl.pallas_call(
    _tc_loss_body,
    grid=(_GRID,),
    in_specs=[
        pl.BlockSpec((_BT, _EMBED), lambda i: (i, 0)),
        pl.BlockSpec((_BT, 128), lambda i: (i, 0)),
        pl.BlockSpec((_BT, 1), lambda i: (i, 0)),
        pl.BlockSpec((_NUM_SAMPLED, 128), lambda i: (0, 0)),
        pl.BlockSpec((1, _NUM_SAMPLED), lambda i: (0, 0)),
    ],
    out_specs=pl.BlockSpec(memory_space=pltpu.SMEM),
    out_shape=jax.ShapeDtypeStruct((1, 1), jnp.float32),
)


def kernel(predictions, labels, weights, biases):
    u = jax.random.uniform(jax.random.key(42), (_NUM_SAMPLED,), dtype=jnp.float32)
    sampled = jnp.clip(
        (jnp.exp(u * jnp.log(_VOCAB + 1.0)) - 1.0).astype(jnp.int32),
        0, _VOCAB - 1)

    labels_i32 = labels.astype(jnp.int32).reshape(-1)
    wtab = _transpose(weights.T, biases.reshape(1, _VOCAB))
    samp_w, true_w = _sc_gather()(
        wtab,
        sampled.reshape(_NUM_SAMPLED // _CHUNK, _CHUNK),
        labels_i32.reshape(_BATCH // _CHUNK, _CHUNK),
    )
    loss = _tc_loss(
        predictions,
        true_w,
        labels_i32.reshape(_BATCH, 1),
        samp_w,
        sampled.reshape(1, _NUM_SAMPLED),
    )
    return loss[0, 0]


# speedup vs baseline: 2.2702x; 1.0475x over previous
import functools
import math

import jax
import jax.numpy as jnp
from jax import lax
from jax.experimental import pallas as pl
from jax.experimental.pallas import tpu as pltpu
from jax.experimental.pallas import tpu_sc as plsc

_VOCAB = 1000000
_EMBED = 64
_NUM_SAMPLED = 8192
_BATCH = 4096

_NC = 2
_NS = 16
_NW = _NC * _NS
_CHUNK = 128
_L = 16

_SAMP_PER_W = _NUM_SAMPLED // _NW
_TRUE_PER_W = _BATCH // _NW
_SAMP_CHUNKS = _SAMP_PER_W // _CHUNK
_TRUE_CHUNKS = _TRUE_PER_W // _CHUNK

_TB = 2048
_TGRID = -(-_VOCAB // _TB)
_TROWS = _TGRID * _TB

_INV_LOG_V1 = 1.0 / math.log(float(_VOCAB) + 1.0)


def _transpose_body(x_ref, b_ref, o_ref):
    x = x_ref[...].astype(jnp.bfloat16)
    eye = (lax.broadcasted_iota(jnp.int32, (_EMBED, _EMBED), 0)
           == lax.broadcasted_iota(jnp.int32, (_EMBED, _EMBED), 1)
           ).astype(jnp.bfloat16)
    xt = lax.dot_general(x, eye, (((0,), (0,)), ((), ())),
                         preferred_element_type=jnp.float32)
    bcol = lax.dot_general(b_ref[...], jnp.ones((1, 1), jnp.float32),
                           (((0,), (0,)), ((), ())),
                           preferred_element_type=jnp.float32)
    o_ref[...] = jnp.concatenate(
        [xt, bcol, jnp.zeros((_TB, 63), jnp.float32)], axis=1)


_transpose = pl.pallas_call(
    _transpose_body,
    grid=(_TGRID,),
    in_specs=[
        pl.BlockSpec((_EMBED, _TB), lambda j: (0, j)),
        pl.BlockSpec((1, _TB), lambda j: (0, j)),
    ],
    out_specs=pl.BlockSpec((_TB, 128), lambda j: (j, 0)),
    out_shape=jax.ShapeDtypeStruct((_TROWS, 128), jnp.float32),
)


def _sc_gather_body(w_hbm, sidx_hbm, tidx_hbm,
                    sw_hbm, tw_hbm,
                    idx_s, idx_t, rows_s, rows_t, sem):
    wid = lax.axis_index("s") * _NC + lax.axis_index("c")
    pltpu.sync_copy(sidx_hbm.at[pl.ds(wid * _SAMP_CHUNKS, _SAMP_CHUNKS)], idx_s)
    pltpu.sync_copy(tidx_hbm.at[pl.ds(wid * _TRUE_CHUNKS, _TRUE_CHUNKS)], idx_t)
    copies = []
    for j in range(_SAMP_CHUNKS):
        copies.append(pltpu.async_copy(
            w_hbm.at[idx_s.at[j]], rows_s.at[pl.ds(j * _CHUNK, _CHUNK)], sem))
    for j in range(_TRUE_CHUNKS):
        copies.append(pltpu.async_copy(
            w_hbm.at[idx_t.at[j]], rows_t.at[pl.ds(j * _CHUNK, _CHUNK)], sem))
    for c in copies:
        c.wait()
    pltpu.sync_copy(rows_s, sw_hbm.at[pl.ds(wid * _SAMP_PER_W, _SAMP_PER_W)])
    pltpu.sync_copy(rows_t, tw_hbm.at[pl.ds(wid * _TRUE_PER_W, _TRUE_PER_W)])


@functools.cache
def _sc_gather():
  return pl.kernel(
    _sc_gather_body,
    out_type=(
        jax.ShapeDtypeStruct((_NUM_SAMPLED, 128), jnp.float32),
        jax.ShapeDtypeStruct((_BATCH, 128), jnp.float32),
    ),
    mesh=plsc.VectorSubcoreMesh(core_axis_name="c", subcore_axis_name="s"),
    scratch_types=[
        pltpu.VMEM((_SAMP_CHUNKS, _CHUNK), jnp.int32),
        pltpu.VMEM((_TRUE_CHUNKS, _CHUNK), jnp.int32),
        pltpu.VMEM((_SAMP_PER_W, 128), jnp.float32),
        pltpu.VMEM((_TRUE_PER_W, 128), jnp.float32),
        pltpu.SemaphoreType.DMA,
    ],
    compiler_params=pltpu.CompilerParams(
        use_tc_tiling_on_sc=True, needs_layout_passes=False),
  )


_BT = 256
_GRID = _BATCH // _BT


def _log1p_small(x):
    return x * (1.0 - x * (1 / 2 - x * (1 / 3 - x * (1 / 4 - x * (
        1 / 5 - x * (1 / 6 - x * (1 / 7)))))))


def _expm1_neg(y):
    small = y * (1.0 + y * (1 / 2 + y * (1 / 6 + y * (1 / 24 + y * (1 / 120)))))
    return jnp.where(y > -0.03, small, jnp.exp(y) - 1.0)


def _log_expected(idsf):
    p = jnp.log((idsf + 2.0) / (idsf + 1.0)) * _INV_LOG_V1
    return jnp.log(-_expm1_neg(_NUM_SAMPLED * _log1p_small(-p)))


def _tc_loss_body(pred_ref, tw_ref, lab_ref, sw_ref, sid_ref, out_ref):
    i = pl.program_id(0)
    pred = pred_ref[...]
    pred_aug = jnp.concatenate(
        [pred, jnp.ones((_BT, 1), jnp.float32)], axis=1)
    sw = sw_ref[...][:, :_EMBED + 1]
    logits = lax.dot_general(
        pred_aug, sw, (((1,), (1,)), ((), ())),
        preferred_element_type=jnp.float32)
    sidf = sid_ref[...].astype(jnp.float32)
    logits = logits - _log_expected(sidf)
    labf = lab_ref[...].astype(jnp.float32)
    tw = tw_ref[...][:, :_EMBED + 1]
    tl = (jnp.sum(pred_aug * tw, axis=1, keepdims=True)
          - _log_expected(labf))
    m = jnp.maximum(jnp.max(logits, axis=1, keepdims=True), tl)
    se = (jnp.sum(jnp.exp(logits - m), axis=1, keepdims=True)
          + jnp.exp(tl - m))
    per_ex = m + jnp.log(se) - tl
    part = jnp.sum(per_ex) * (1.0 / _BATCH)

    @pl.when(i == 0)
    def _():
        out_ref[0, 0] = 0.0

    out_ref[0, 0] += part


_tc_loss = pl.pallas_call(
    _tc_loss_body,
    grid=(_GRID,),
    in_specs=[
        pl.BlockSpec((_BT, _EMBED), lambda i: (i, 0)),
        pl.BlockSpec((_BT, 128), lambda i: (i, 0)),
        pl.BlockSpec((_BT, 1), lambda i: (i, 0)),
        pl.BlockSpec((_NUM_SAMPLED, 128), lambda i: (0, 0)),
        pl.BlockSpec((1, _NUM_SAMPLED), lambda i: (0, 0)),
    ],
    out_specs=pl.BlockSpec(memory_space=pltpu.SMEM),
    out_shape=jax.ShapeDtypeStruct((1, 1), jnp.float32),
)


def kernel(predictions, labels, weights, biases):
    u = jax.random.uniform(jax.random.key(42), (_NUM_SAMPLED,), dtype=jnp.float32)
    sampled = jnp.clip(
        (jnp.exp(u * jnp.log(_VOCAB + 1.0)) - 1.0).astype(jnp.int32),
        0, _VOCAB - 1)

    labels_i32 = labels.astype(jnp.int32).reshape(-1)
    wtab = _transpose(weights.T, biases.reshape(1, _VOCAB))
    samp_w, true_w = _sc_gather()(
        wtab,
        sampled.reshape(_NUM_SAMPLED // _CHUNK, _CHUNK),
        labels_i32.reshape(_BATCH // _CHUNK, _CHUNK),
    )
    loss = _tc_loss(
        predictions,
        true_w,
        labels_i32.reshape(_BATCH, 1),
        samp_w,
        sampled.reshape(1, _NUM_SAMPLED),
    )
    return loss[0, 0]


# speedup vs baseline: 3.1792x; 1.4004x over previous
import functools
import math

import jax
import jax.numpy as jnp
from jax import lax
from jax.experimental import pallas as pl
from jax.experimental.pallas import tpu as pltpu
from jax.experimental.pallas import tpu_sc as plsc

_VOCAB = 1000000
_EMBED = 64
_NUM_SAMPLED = 8192
_BATCH = 4096

_NC = 2
_NS = 16
_NW = _NC * _NS
_CHUNK = 128
_L = 16

_SAMP_PER_W = _NUM_SAMPLED // _NW
_TRUE_PER_W = _BATCH // _NW
_SAMP_CHUNKS = _SAMP_PER_W // _CHUNK
_TRUE_CHUNKS = _TRUE_PER_W // _CHUNK

_TB = 8192
_TGRID = -(-_VOCAB // _TB)
_TROWS = _TGRID * _TB

_INV_LOG_V1 = 1.0 / math.log(float(_VOCAB) + 1.0)


def _transpose_body(x_ref, b_ref, o_ref):
    x = x_ref[...].astype(jnp.bfloat16)
    eye = (lax.broadcasted_iota(jnp.int32, (_EMBED, _EMBED), 0)
           == lax.broadcasted_iota(jnp.int32, (_EMBED, _EMBED), 1)
           ).astype(jnp.bfloat16)
    xt = lax.dot_general(x, eye, (((0,), (0,)), ((), ())),
                         preferred_element_type=jnp.float32)
    bcol = lax.dot_general(b_ref[...], jnp.ones((1, 1), jnp.float32),
                           (((0,), (0,)), ((), ())),
                           preferred_element_type=jnp.float32)
    o_ref[...] = jnp.concatenate(
        [xt, bcol, jnp.zeros((_TB, 63), jnp.float32)], axis=1)


_transpose = pl.pallas_call(
    _transpose_body,
    grid=(_TGRID,),
    in_specs=[
        pl.BlockSpec((_EMBED, _TB), lambda j: (0, j)),
        pl.BlockSpec((1, _TB), lambda j: (0, j)),
    ],
    out_specs=pl.BlockSpec((_TB, 128), lambda j: (j, 0)),
    out_shape=jax.ShapeDtypeStruct((_TROWS, 128), jnp.float32),
)


def _sc_gather_body(w_hbm, sidx_hbm, tidx_hbm,
                    sw_hbm, tw_hbm,
                    idx_s, idx_t, rows_s, rows_t, sem):
    wid = lax.axis_index("s") * _NC + lax.axis_index("c")
    pltpu.sync_copy(sidx_hbm.at[pl.ds(wid * _SAMP_CHUNKS, _SAMP_CHUNKS)], idx_s)
    pltpu.sync_copy(tidx_hbm.at[pl.ds(wid * _TRUE_CHUNKS, _TRUE_CHUNKS)], idx_t)
    copies = []
    for j in range(_SAMP_CHUNKS):
        copies.append(pltpu.async_copy(
            w_hbm.at[idx_s.at[j]], rows_s.at[pl.ds(j * _CHUNK, _CHUNK)], sem))
    for j in range(_TRUE_CHUNKS):
        copies.append(pltpu.async_copy(
            w_hbm.at[idx_t.at[j]], rows_t.at[pl.ds(j * _CHUNK, _CHUNK)], sem))
    for c in copies:
        c.wait()
    pltpu.sync_copy(rows_s, sw_hbm.at[pl.ds(wid * _SAMP_PER_W, _SAMP_PER_W)])
    pltpu.sync_copy(rows_t, tw_hbm.at[pl.ds(wid * _TRUE_PER_W, _TRUE_PER_W)])


@functools.cache
def _sc_gather():
  return pl.kernel(
    _sc_gather_body,
    out_type=(
        jax.ShapeDtypeStruct((_NUM_SAMPLED, 128), jnp.float32),
        jax.ShapeDtypeStruct((_BATCH, 128), jnp.float32),
    ),
    mesh=plsc.VectorSubcoreMesh(core_axis_name="c", subcore_axis_name="s"),
    scratch_types=[
        pltpu.VMEM((_SAMP_CHUNKS, _CHUNK), jnp.int32),
        pltpu.VMEM((_TRUE_CHUNKS, _CHUNK), jnp.int32),
        pltpu.VMEM((_SAMP_PER_W, 128), jnp.float32),
        pltpu.VMEM((_TRUE_PER_W, 128), jnp.float32),
        pltpu.SemaphoreType.DMA,
    ],
    compiler_params=pltpu.CompilerParams(
        use_tc_tiling_on_sc=True, needs_layout_passes=False),
  )


_BT = 256
_GRID = _BATCH // _BT


def _log1p_small(x):
    return x * (1.0 - x * (1 / 2 - x * (1 / 3 - x * (1 / 4 - x * (
        1 / 5 - x * (1 / 6 - x * (1 / 7)))))))


def _expm1_neg(y):
    small = y * (1.0 + y * (1 / 2 + y * (1 / 6 + y * (1 / 24 + y * (1 / 120)))))
    return jnp.where(y > -0.03, small, jnp.exp(y) - 1.0)


def _log_expected(idsf):
    p = jnp.log((idsf + 2.0) / (idsf + 1.0)) * _INV_LOG_V1
    return jnp.log(-_expm1_neg(_NUM_SAMPLED * _log1p_small(-p)))


def _tc_loss_body(pred_ref, tw_ref, lab_ref, sw_ref, sid_ref, out_ref):
    i = pl.program_id(0)
    pred = pred_ref[...]
    pred_aug = jnp.concatenate(
        [pred, jnp.ones((_BT, 1), jnp.float32)], axis=1)
    sw = sw_ref[...][:, :_EMBED + 1]
    logits = lax.dot_general(
        pred_aug, sw, (((1,), (1,)), ((), ())),
        preferred_element_type=jnp.float32)
    sidf = sid_ref[...].astype(jnp.float32)
    logits = logits - _log_expected(sidf)
    labf = lab_ref[...].astype(jnp.float32)
    tw = tw_ref[...][:, :_EMBED + 1]
    tl = (jnp.sum(pred_aug * tw, axis=1, keepdims=True)
          - _log_expected(labf))
    m = jnp.maximum(jnp.max(logits, axis=1, keepdims=True), tl)
    se = (jnp.sum(jnp.exp(logits - m), axis=1, keepdims=True)
          + jnp.exp(tl - m))
    per_ex = m + jnp.log(se) - tl
    part = jnp.sum(per_ex) * (1.0 / _BATCH)

    @pl.when(i == 0)
    def _():
        out_ref[0, 0] = 0.0

    out_ref[0, 0] += part


_tc_loss = pl.pallas_call(
    _tc_loss_body,
    grid=(_GRID,),
    in_specs=[
        pl.BlockSpec((_BT, _EMBED), lambda i: (i, 0)),
        pl.BlockSpec((_BT, 128), lambda i: (i, 0)),
        pl.BlockSpec((_BT, 1), lambda i: (i, 0)),
        pl.BlockSpec((_NUM_SAMPLED, 128), lambda i: (0, 0)),
        pl.BlockSpec((1, _NUM_SAMPLED), lambda i: (0, 0)),
    ],
    out_specs=pl.BlockSpec(memory_space=pltpu.SMEM),
    out_shape=jax.ShapeDtypeStruct((1, 1), jnp.float32),
)


def kernel(predictions, labels, weights, biases):
    u = jax.random.uniform(jax.random.key(42), (_NUM_SAMPLED,), dtype=jnp.float32)
    sampled = jnp.clip(
        (jnp.exp(u * jnp.log(_VOCAB + 1.0)) - 1.0).astype(jnp.int32),
        0, _VOCAB - 1)

    labels_i32 = labels.astype(jnp.int32).reshape(-1)
    wtab = _transpose(weights.T, biases.reshape(1, _VOCAB))
    samp_w, true_w = _sc_gather()(
        wtab,
        sampled.reshape(_NUM_SAMPLED // _CHUNK, _CHUNK),
        labels_i32.reshape(_BATCH // _CHUNK, _CHUNK),
    )
    loss = _tc_loss(
        predictions,
        true_w,
        labels_i32.reshape(_BATCH, 1),
        samp_w,
        sampled.reshape(1, _NUM_SAMPLED),
    )
    return loss[0, 0]


# speedup vs baseline: 3.2522x; 1.0230x over previous
import functools
import math

import jax
import jax.numpy as jnp
from jax import lax
from jax.experimental import pallas as pl
from jax.experimental.pallas import tpu as pltpu
from jax.experimental.pallas import tpu_sc as plsc

_VOCAB = 1000000
_EMBED = 64
_NUM_SAMPLED = 8192
_BATCH = 4096

_NC = 2
_NS = 16
_NW = _NC * _NS
_CHUNK = 128
_L = 16

_SAMP_PER_W = _NUM_SAMPLED // _NW
_TRUE_PER_W = _BATCH // _NW
_SAMP_CHUNKS = _SAMP_PER_W // _CHUNK
_TRUE_CHUNKS = _TRUE_PER_W // _CHUNK

_TB = 16384
_TGRID = -(-_VOCAB // _TB)
_TROWS = _TGRID * _TB

_INV_LOG_V1 = 1.0 / math.log(float(_VOCAB) + 1.0)


def _transpose_body(x_ref, b_ref, o_ref):
    x = x_ref[...].astype(jnp.bfloat16)
    eye = (lax.broadcasted_iota(jnp.int32, (_EMBED, _EMBED), 0)
           == lax.broadcasted_iota(jnp.int32, (_EMBED, _EMBED), 1)
           ).astype(jnp.bfloat16)
    xt = lax.dot_general(x, eye, (((0,), (0,)), ((), ())),
                         preferred_element_type=jnp.float32)
    bcol = lax.dot_general(b_ref[...], jnp.ones((1, 1), jnp.float32),
                           (((0,), (0,)), ((), ())),
                           preferred_element_type=jnp.float32)
    o_ref[...] = jnp.concatenate(
        [xt, bcol, jnp.zeros((_TB, 63), jnp.float32)], axis=1)


_transpose = pl.pallas_call(
    _transpose_body,
    grid=(_TGRID,),
    in_specs=[
        pl.BlockSpec((_EMBED, _TB), lambda j: (0, j)),
        pl.BlockSpec((1, _TB), lambda j: (0, j)),
    ],
    out_specs=pl.BlockSpec((_TB, 128), lambda j: (j, 0)),
    out_shape=jax.ShapeDtypeStruct((_TROWS, 128), jnp.float32),
)


def _sc_gather_body(w_hbm, sidx_hbm, tidx_hbm,
                    sw_hbm, tw_hbm,
                    idx_s, idx_t, rows_s, rows_t, sem):
    wid = lax.axis_index("s") * _NC + lax.axis_index("c")
    pltpu.sync_copy(sidx_hbm.at[pl.ds(wid * _SAMP_CHUNKS, _SAMP_CHUNKS)], idx_s)
    pltpu.sync_copy(tidx_hbm.at[pl.ds(wid * _TRUE_CHUNKS, _TRUE_CHUNKS)], idx_t)
    copies = []
    for j in range(_SAMP_CHUNKS):
        copies.append(pltpu.async_copy(
            w_hbm.at[idx_s.at[j]], rows_s.at[pl.ds(j * _CHUNK, _CHUNK)], sem))
    for j in range(_TRUE_CHUNKS):
        copies.append(pltpu.async_copy(
            w_hbm.at[idx_t.at[j]], rows_t.at[pl.ds(j * _CHUNK, _CHUNK)], sem))
    for c in copies:
        c.wait()
    pltpu.sync_copy(rows_s, sw_hbm.at[pl.ds(wid * _SAMP_PER_W, _SAMP_PER_W)])
    pltpu.sync_copy(rows_t, tw_hbm.at[pl.ds(wid * _TRUE_PER_W, _TRUE_PER_W)])


@functools.cache
def _sc_gather():
  return pl.kernel(
    _sc_gather_body,
    out_type=(
        jax.ShapeDtypeStruct((_NUM_SAMPLED, 128), jnp.float32),
        jax.ShapeDtypeStruct((_BATCH, 128), jnp.float32),
    ),
    mesh=plsc.VectorSubcoreMesh(core_axis_name="c", subcore_axis_name="s"),
    scratch_types=[
        pltpu.VMEM((_SAMP_CHUNKS, _CHUNK), jnp.int32),
        pltpu.VMEM((_TRUE_CHUNKS, _CHUNK), jnp.int32),
        pltpu.VMEM((_SAMP_PER_W, 128), jnp.float32),
        pltpu.VMEM((_TRUE_PER_W, 128), jnp.float32),
        pltpu.SemaphoreType.DMA,
    ],
    compiler_params=pltpu.CompilerParams(
        use_tc_tiling_on_sc=True, needs_layout_passes=False),
  )


_BT = 256
_GRID = _BATCH // _BT


def _log1p_small(x):
    return x * (1.0 - x * (1 / 2 - x * (1 / 3 - x * (1 / 4 - x * (
        1 / 5 - x * (1 / 6 - x * (1 / 7)))))))


def _expm1_neg(y):
    small = y * (1.0 + y * (1 / 2 + y * (1 / 6 + y * (1 / 24 + y * (1 / 120)))))
    return jnp.where(y > -0.03, small, jnp.exp(y) - 1.0)


def _log_expected(idsf):
    p = jnp.log((idsf + 2.0) / (idsf + 1.0)) * _INV_LOG_V1
    return jnp.log(-_expm1_neg(_NUM_SAMPLED * _log1p_small(-p)))


def _tc_loss_body(pred_ref, tw_ref, lab_ref, sw_ref, sid_ref, out_ref):
    i = pl.program_id(0)
    pred = pred_ref[...]
    pred_aug = jnp.concatenate(
        [pred, jnp.ones((_BT, 1), jnp.float32)], axis=1)
    sw = sw_ref[...][:, :_EMBED + 1]
    logits = lax.dot_general(
        pred_aug, sw, (((1,), (1,)), ((), ())),
        preferred_element_type=jnp.float32)
    sidf = sid_ref[...].astype(jnp.float32)
    logits = logits - _log_expected(sidf)
    labf = lab_ref[...].astype(jnp.float32)
    tw = tw_ref[...][:, :_EMBED + 1]
    tl = (jnp.sum(pred_aug * tw, axis=1, keepdims=True)
          - _log_expected(labf))
    m = jnp.maximum(jnp.max(logits, axis=1, keepdims=True), tl)
    se = (jnp.sum(jnp.exp(logits - m), axis=1, keepdims=True)
          + jnp.exp(tl - m))
    per_ex = m + jnp.log(se) - tl
    part = jnp.sum(per_ex) * (1.0 / _BATCH)

    @pl.when(i == 0)
    def _():
        out_ref[0, 0] = 0.0

    out_ref[0, 0] += part


_tc_loss = pl.pallas_call(
    _tc_loss_body,
    grid=(_GRID,),
    in_specs=[
        pl.BlockSpec((_BT, _EMBED), lambda i: (i, 0)),
        pl.BlockSpec((_BT, 128), lambda i: (i, 0)),
        pl.BlockSpec((_BT, 1), lambda i: (i, 0)),
        pl.BlockSpec((_NUM_SAMPLED, 128), lambda i: (0, 0)),
        pl.BlockSpec((1, _NUM_SAMPLED), lambda i: (0, 0)),
    ],
    out_specs=pl.BlockSpec(memory_space=pltpu.SMEM),
    out_shape=jax.ShapeDtypeStruct((1, 1), jnp.float32),
)


def kernel(predictions, labels, weights, biases):
    u = jax.random.uniform(jax.random.key(42), (_NUM_SAMPLED,), dtype=jnp.float32)
    sampled = jnp.clip(
        (jnp.exp(u * jnp.log(_VOCAB + 1.0)) - 1.0).astype(jnp.int32),
        0, _VOCAB - 1)

    labels_i32 = labels.astype(jnp.int32).reshape(-1)
    wtab = _transpose(weights.T, biases.reshape(1, _VOCAB))
    samp_w, true_w = _sc_gather()(
        wtab,
        sampled.reshape(_NUM_SAMPLED // _CHUNK, _CHUNK),
        labels_i32.reshape(_BATCH // _CHUNK, _CHUNK),
    )
    loss = _tc_loss(
        predictions,
        true_w,
        labels_i32.reshape(_BATCH, 1),
        samp_w,
        sampled.reshape(1, _NUM_SAMPLED),
    )
    return loss[0, 0]
